# Initial kernel scaffold; baseline (speedup 1.0000x reference)
import jax
import jax.numpy as jnp
from jax.experimental import pallas as pl


def kernel(x, edge_index, batch, W1, b1, p1, W2, b2, p2, W3, b3, p3, W4, b4, p4, Wl1, bl1, Wl2, bl2, Wl3, bl3):
    raise NotImplementedError("write your pallas kernel here")


# speedup vs baseline: 16.0216x; 16.0216x over previous
import functools
import jax
import jax.numpy as jnp
from jax import lax
from jax.experimental import pallas as pl
from jax.experimental.pallas import tpu as pltpu, tpu_sc as plsc

N = 10000
NPAD = 10240
E = 320000
D = 128
NG = 64
HOPS = 3
ALPHA = 0.1
KS = (5000, 2500, 1250, 625)
DUMMY = N

NTILES = 32
EPT = E // NTILES
CH = 80
NCH = EPT // CH
EPT1 = E // 16
NCH1 = EPT1 // CH
NSL = NPAD // 16

_MESH = dict(
    mesh=plsc.VectorSubcoreMesh(core_axis_name="c", subcore_axis_name="s"),
    compiler_params=pltpu.CompilerParams(needs_layout_passes=False),
)
_INT_MIN = -2147483648


@functools.partial(
    pl.kernel,
    out_type=[
        jax.ShapeDtypeStruct((E,), jnp.float32),
        jax.ShapeDtypeStruct((E,), jnp.int32),
        jax.ShapeDtypeStruct((2, NPAD), jnp.float32),
    ],
    scratch_types=[
        pltpu.VMEM((NPAD,), jnp.float32),
        pltpu.VMEM((CH,), jnp.int32),
        pltpu.VMEM((CH,), jnp.int32),
        pltpu.VMEM((CH,), jnp.float32),
        pltpu.VMEM((CH,), jnp.float32),
        pltpu.VMEM((CH,), jnp.int32),
        pltpu.VMEM_SHARED((NPAD,), jnp.float32),
    ],
    **_MESH,
)
def _sc_edge_prep(nv_hbm, src_hbm, dst_hbm, ap_hbm, zv_hbm,
                  alive_hbm, dred_hbm, deg_hbm,
                  nv_v, srcb, dstb, apb, alb, drb, deg_sp):
    cid = lax.axis_index("c")
    sid = lax.axis_index("s")
    wid = cid * 16 + sid

    pltpu.sync_copy(nv_hbm, nv_v)
    pltpu.sync_copy(zv_hbm, deg_sp.at[pl.ds(sid * NSL, NSL)])
    plsc.subcore_barrier()

    def chunk(c, _):
        base = wid * EPT + c * CH
        pltpu.sync_copy(src_hbm.at[pl.ds(base, CH)], srcb)
        pltpu.sync_copy(dst_hbm.at[pl.ds(base, CH)], dstb)
        pltpu.sync_copy(ap_hbm.at[pl.ds(base, CH)], apb)
        for j in range(CH // 16):
            sl = pl.ds(j * 16, 16)
            sv = srcb[sl]
            dv = dstb[sl]
            ga = plsc.load_gather(nv_v, [sv])
            gb = plsc.load_gather(nv_v, [dv])
            al = apb[sl] * ga * gb
            alb[sl] = al
            drb[sl] = jnp.where(al > 0.0, dv, DUMMY)
        pltpu.sync_copy(alb, alive_hbm.at[pl.ds(base, CH)])
        pltpu.sync_copy(drb, dred_hbm.at[pl.ds(base, CH)])
        pltpu.sync_copy(alb, deg_sp.at[dstb], add=True)
        return _

    lax.fori_loop(0, NCH, chunk, 0)
    plsc.subcore_barrier()
    pltpu.sync_copy(deg_sp.at[pl.ds(sid * NSL, NSL)],
                    deg_hbm.at[cid, pl.ds(sid * NSL, NSL)])


@functools.partial(
    pl.kernel,
    out_type=[jax.ShapeDtypeStruct((2, NPAD, D), jnp.float32)],
    scratch_types=[
        pltpu.VMEM((CH,), jnp.int32),
        pltpu.VMEM((CH,), jnp.int32),
        pltpu.VMEM((CH, D), jnp.float32),
        pltpu.SemaphoreType.DMA,
        pltpu.VMEM_SHARED((NPAD, D), jnp.float32),
    ],
    **_MESH,
)
def _sc_feat_agg(y_hbm, src_hbm, dred_hbm, zr_hbm, agg_hbm,
                 idxs, idxd, rows, sem, acc_sp):
    cid = lax.axis_index("c")
    sid = lax.axis_index("s")
    wid = cid * 16 + sid

    for t in range(NSL // CH):
        pltpu.sync_copy(zr_hbm, acc_sp.at[pl.ds(sid * NSL + t * CH, CH)])
    plsc.subcore_barrier()

    def chunk(c, _):
        base = wid * EPT + c * CH
        pltpu.sync_copy(src_hbm.at[pl.ds(base, CH)], idxs)
        pltpu.sync_copy(dred_hbm.at[pl.ds(base, CH)], idxd)
        pltpu.async_copy(y_hbm.at[idxs], rows, sem).wait()
        pltpu.sync_copy(rows, acc_sp.at[idxd], add=True)
        return _

    lax.fori_loop(0, NCH, chunk, 0)
    plsc.subcore_barrier()
    pltpu.sync_copy(acc_sp.at[pl.ds(sid * NSL, NSL)],
                    agg_hbm.at[cid, pl.ds(sid * NSL, NSL)])


@functools.partial(
    pl.kernel,
    out_type=[jax.ShapeDtypeStruct((NPAD,), jnp.float32)],
    scratch_types=[
        pltpu.VMEM((NPAD,), jnp.float32),
        pltpu.VMEM((NPAD,), jnp.float32),
        pltpu.VMEM((EPT1,), jnp.float32),
        pltpu.VMEM((EPT1,), jnp.int32),
        pltpu.VMEM((NCH1, CH), jnp.int32),
        pltpu.VMEM((CH,), jnp.float32),
        pltpu.VMEM((CH,), jnp.float32),
        pltpu.VMEM((NSL,), jnp.float32),
        pltpu.VMEM((NSL,), jnp.float32),
        pltpu.VMEM((NSL,), jnp.float32),
        pltpu.VMEM((NSL,), jnp.float32),
        pltpu.VMEM_SHARED((NPAD,), jnp.float32),
        pltpu.VMEM_SHARED((NPAD,), jnp.float32),
    ],
    **_MESH,
)
def _sc_sprop(s0_hbm, dinv_hbm, dinv2_hbm, src_hbm, dst_hbm, alive_hbm, zv_hbm,
              s_hbm,
              dinv_v, s_v, coef_v, src_v, dst_v, alb, valb, s0b, d2b, propb, snewb,
              prop_sp, s_sp):
    cid = lax.axis_index("c")
    sid = lax.axis_index("s")

    @pl.when(cid == 0)
    def _body():
            nsl = pl.ds(sid * NSL, NSL)
            pltpu.sync_copy(dinv_hbm, dinv_v)
            pltpu.sync_copy(s0_hbm, s_v)
            pltpu.sync_copy(s0_hbm.at[nsl], s0b)
            pltpu.sync_copy(dinv2_hbm.at[nsl], d2b)
            pltpu.sync_copy(zv_hbm, prop_sp.at[nsl])

            def stage(c, _):
                base = sid * EPT1 + c * CH
                pltpu.sync_copy(src_hbm.at[pl.ds(base, CH)],
                                src_v.at[pl.ds(c * CH, CH)])
                pltpu.sync_copy(dst_hbm.at[pl.ds(base, CH)], dst_v.at[c])
                pltpu.sync_copy(alive_hbm.at[pl.ds(base, CH)], alb)
                for j in range(CH // 16):
                    sl = pl.ds(c * CH + j * 16, 16)
                    gs = plsc.load_gather(dinv_v, [src_v[sl]])
                    gd = plsc.load_gather(dinv_v, [dst_v[c, pl.ds(j * 16, 16)]])
                    coef_v[sl] = alb[pl.ds(j * 16, 16)] * gs * gd
                return _

            lax.fori_loop(0, NCH1, stage, 0)
            plsc.subcore_barrier()

            for h in range(HOPS):
                def scat(c, _):
                    for j in range(CH // 16):
                        sl = pl.ds(c * CH + j * 16, 16)
                        sv = plsc.load_gather(s_v, [src_v[sl]])
                        valb[pl.ds(j * 16, 16)] = sv * coef_v[sl]
                    pltpu.sync_copy(valb, prop_sp.at[dst_v.at[c]], add=True)
                    return _

                lax.fori_loop(0, NCH1, scat, 0)
                plsc.subcore_barrier()
                pltpu.sync_copy(prop_sp.at[nsl], propb)
                for i in range(NSL // 16):
                    sl16 = pl.ds(i * 16, 16)
                    sv = s_v[pl.ds(sid * NSL + i * 16, 16)]
                    snewb[sl16] = ALPHA * s0b[sl16] + (1.0 - ALPHA) * (
                        propb[sl16] + sv * d2b[sl16])
                pltpu.sync_copy(snewb, s_sp.at[nsl])
                if h < HOPS - 1:
                    pltpu.sync_copy(zv_hbm, prop_sp.at[nsl])
                    plsc.subcore_barrier()
                    pltpu.sync_copy(s_sp, s_v)
                else:
                    pltpu.sync_copy(snewb, s_hbm.at[nsl])


_RB = 1024
_NRB = NPAD // _RB


def _tc_matmul(x, W):
    def body(x_ref, w_ref, o_ref):
        o_ref[...] = jnp.dot(x_ref[...], w_ref[...],
                             preferred_element_type=jnp.float32)
    return pl.pallas_call(
        body,
        grid=(_NRB,),
        in_specs=[pl.BlockSpec((_RB, D), lambda i: (i, 0)),
                  pl.BlockSpec((D, D), lambda i: (0, 0))],
        out_specs=pl.BlockSpec((_RB, D), lambda i: (i, 0)),
        out_shape=jax.ShapeDtypeStruct((NPAD, D), jnp.float32),
    )(x, W)


def _tc_mid(deg_p, xw):
    def body(dp_ref, xw_ref, y_ref, di_ref, d2_ref):
        deg = dp_ref[0] + dp_ref[1] + 1.0
        dinv = lax.rsqrt(deg)
        di_ref[...] = dinv
        d2_ref[...] = 1.0 / deg
        y_ref[...] = xw_ref[...] * dinv
    return pl.pallas_call(
        body,
        grid=(_NRB,),
        in_specs=[pl.BlockSpec((2, _RB, 1), lambda i: (0, i, 0)),
                  pl.BlockSpec((_RB, D), lambda i: (i, 0))],
        out_specs=[pl.BlockSpec((_RB, D), lambda i: (i, 0)),
                   pl.BlockSpec((_RB, 1), lambda i: (i, 0)),
                   pl.BlockSpec((_RB, 1), lambda i: (i, 0))],
        out_shape=[jax.ShapeDtypeStruct((NPAD, D), jnp.float32),
                   jax.ShapeDtypeStruct((NPAD, 1), jnp.float32),
                   jax.ShapeDtypeStruct((NPAD, 1), jnp.float32)],
    )(deg_p, xw)


def _tc_post(agg_p, xw, dinv, dinv2, nv, b, p):
    def body(ag_ref, xw_ref, di_ref, d2_ref, nv_ref, b_ref, p_ref,
             xn_ref, s0_ref):
        agg = ag_ref[0] + ag_ref[1]
        xn = jax.nn.relu(agg * di_ref[...] + xw_ref[...] * d2_ref[...]
                         + b_ref[...]) * nv_ref[...]
        xn_ref[...] = xn
        s0_ref[...] = jnp.dot(xn, p_ref[...],
                              preferred_element_type=jnp.float32)
    return pl.pallas_call(
        body,
        grid=(_NRB,),
        in_specs=[pl.BlockSpec((2, _RB, D), lambda i: (0, i, 0)),
                  pl.BlockSpec((_RB, D), lambda i: (i, 0)),
                  pl.BlockSpec((_RB, 1), lambda i: (i, 0)),
                  pl.BlockSpec((_RB, 1), lambda i: (i, 0)),
                  pl.BlockSpec((_RB, 1), lambda i: (i, 0)),
                  pl.BlockSpec((1, D), lambda i: (0, 0)),
                  pl.BlockSpec((D, 1), lambda i: (0, 0))],
        out_specs=[pl.BlockSpec((_RB, D), lambda i: (i, 0)),
                   pl.BlockSpec((_RB, 1), lambda i: (i, 0))],
        out_shape=[jax.ShapeDtypeStruct((NPAD, D), jnp.float32),
                   jax.ShapeDtypeStruct((NPAD, 1), jnp.float32)],
    )(agg_p, xw, dinv, dinv2, nv, b, p)


def _tc_pool(s, nv, xn, k):
    def body(s_ref, nv_ref, xn_ref, nvn_ref, xg_ref):
        s_val = s_ref[...]
        bits = lax.bitcast_convert_type(s_val, jnp.int32)
        key = bits ^ ((bits >> 31) & jnp.int32(0x7FFFFFFF))
        key = jnp.where(nv_ref[...] > 0.0, key, jnp.int32(_INT_MIN))

        def step(i, cur_u):
            cand_u = cur_u | (jnp.int32(1) << (31 - i))
            cand_key = cand_u ^ _INT_MIN
            cnt = jnp.sum((key >= cand_key).astype(jnp.int32))
            return jnp.where(cnt >= k, cand_u, cur_u)

        cur_u = lax.fori_loop(0, 32, step, jnp.int32(0))
        t_key = cur_u ^ _INT_MIN
        nvn = (key >= t_key).astype(jnp.float32)
        nvn_ref[...] = nvn
        xg_ref[...] = xn_ref[...] * jnp.tanh(s_val) * nvn
    return pl.pallas_call(
        body,
        out_shape=[jax.ShapeDtypeStruct((NPAD, 1), jnp.float32),
                   jax.ShapeDtypeStruct((NPAD, D), jnp.float32)],
    )(s, nv, xn)


def _tc_readout(xg, nv, batch):
    def body(xg_ref, nv_ref, b_ref, mx_ref, sm_ref, cnt_ref):
        pid = pl.program_id(0)

        @pl.when(pid == 0)
        def _():
            mx_ref[...] = jnp.full((NG, D), -jnp.inf, jnp.float32)
            sm_ref[...] = jnp.zeros((NG, D), jnp.float32)
            cnt_ref[...] = jnp.zeros((NG, 1), jnp.float32)

        bid = b_ref[...]
        nvb = nv_ref[...]
        xgb = xg_ref[...]
        gids = lax.broadcasted_iota(jnp.int32, (1, NG), 1)
        oh = (bid == gids).astype(jnp.float32) * nvb
        dn = (((0,), (0,)), ((), ()))
        sm_ref[...] += lax.dot_general(oh, xgb, dn,
                                       preferred_element_type=jnp.float32)
        cnt_ref[...] += lax.dot_general(oh, nvb, dn,
                                        preferred_element_type=jnp.float32)

        def gstep(g, _):
            m = jnp.max(jnp.where((bid == g) & (nvb > 0.0), xgb, -jnp.inf),
                        axis=0, keepdims=True)
            mx_ref[pl.ds(g, 1), :] = jnp.maximum(mx_ref[pl.ds(g, 1), :], m)
            return _

        lax.fori_loop(0, NG, gstep, 0)

    return pl.pallas_call(
        body,
        grid=(_NRB,),
        in_specs=[pl.BlockSpec((_RB, D), lambda i: (i, 0)),
                  pl.BlockSpec((_RB, 1), lambda i: (i, 0)),
                  pl.BlockSpec((_RB, 1), lambda i: (i, 0))],
        out_specs=[pl.BlockSpec((NG, D), lambda i: (0, 0)),
                   pl.BlockSpec((NG, D), lambda i: (0, 0)),
                   pl.BlockSpec((NG, 1), lambda i: (0, 0))],
        out_shape=[jax.ShapeDtypeStruct((NG, D), jnp.float32),
                   jax.ShapeDtypeStruct((NG, D), jnp.float32),
                   jax.ShapeDtypeStruct((NG, 1), jnp.float32)],
    )(xg, nv, batch)


def _tc_head(mxs, sms, cnts, Wl1, bl1, Wl2, bl2, Wl3, bl3):
    def body(m1, m2, m3, m4, s1, s2, s3, s4, c1, c2, c3, c4,
             w1, b1, w2, b2, w3, b3, o_ref):
        h = jnp.zeros((NG, 2 * D), jnp.float32)
        for m_ref, s_ref, c_ref in ((m1, s1, c1), (m2, s2, c2),
                                    (m3, s3, c3), (m4, s4, c4)):
            mx = m_ref[...]
            mx = jnp.where(jnp.isfinite(mx), mx, 0.0)
            mean = s_ref[...] / jnp.maximum(c_ref[...], 1.0)
            h = h + jnp.concatenate([mx, mean], axis=1)
        z = jax.nn.relu(jnp.dot(h, w1[...],
                                preferred_element_type=jnp.float32) + b1[...])
        z = jax.nn.relu(jnp.dot(z, w2[...],
                                preferred_element_type=jnp.float32) + b2[...])
        z = jnp.dot(z, w3[...], preferred_element_type=jnp.float32) + b3[...]
        zm = z - jnp.max(z, axis=1, keepdims=True)
        o_ref[...] = zm - jnp.log(jnp.sum(jnp.exp(zm), axis=1, keepdims=True))

    args = list(mxs) + list(sms) + list(cnts) + [
        Wl1, bl1.reshape(1, -1), Wl2, bl2.reshape(1, -1), Wl3, bl3.reshape(1, -1)]
    return pl.pallas_call(
        body,
        out_shape=jax.ShapeDtypeStruct((NG, 10), jnp.float32),
    )(*args)


def kernel(x, edge_index, batch, W1, b1, p1, W2, b2, p2, W3, b3, p3,
           W4, b4, p4, Wl1, bl1, Wl2, bl2, Wl3, bl3):
    src = edge_index[0]
    dst = edge_index[1]
    xs = jnp.pad(x, ((0, NPAD - N), (0, 0)))
    batchp = jnp.pad(batch, (0, NPAD - N)).reshape(NPAD, 1)
    nv_col = jnp.pad(jnp.ones((N, 1), jnp.float32), ((0, NPAD - N), (0, 0)))
    alive = jnp.ones((E,), jnp.float32)
    zv = jnp.zeros((NSL,), jnp.float32)
    zr = jnp.zeros((CH, D), jnp.float32)

    Ws = (W1, W2, W3, W4)
    bs = (b1, b2, b3, b4)
    ps = (p1, p2, p3, p4)
    mxs, sms, cnts = [], [], []

    for l in range(4):
        nv_flat = nv_col.reshape(NPAD)
        alive, dred, deg_p = _sc_edge_prep(nv_flat, src, dst, alive, zv)
        xw = _tc_matmul(xs, Ws[l])
        y, dinv, dinv2 = _tc_mid(deg_p.reshape(2, NPAD, 1), xw)
        (agg_p,) = _sc_feat_agg(y, src, dred, zr)
        xn, s0 = _tc_post(agg_p, xw, dinv, dinv2, nv_col,
                          bs[l].reshape(1, D), ps[l])
        (s,) = _sc_sprop(s0.reshape(NPAD), dinv.reshape(NPAD),
                         dinv2.reshape(NPAD), src, dst, alive, zv)
        nv_col, xg = _tc_pool(s.reshape(NPAD, 1), nv_col, xn, KS[l])
        mx, sm, cnt = _tc_readout(xg, nv_col, batchp)
        mxs.append(mx); sms.append(sm); cnts.append(cnt)
        xs = xg

    return _tc_head(mxs, sms, cnts, Wl1, bl1, Wl2, bl2, Wl3, bl3)


# speedup vs baseline: 21.8549x; 1.3641x over previous
import functools
import jax
import jax.numpy as jnp
from jax import lax
from jax.experimental import pallas as pl
from jax.experimental.pallas import tpu as pltpu, tpu_sc as plsc

N = 10000
NPAD = 10240
E = 320000
D = 128
NG = 64
HOPS = 3
ALPHA = 0.1
KS = (5000, 2500, 1250, 625)
DUMMY = N

NTILES = 32
EPT = E // NTILES
CH = 80
NCH = EPT // CH
EPT1 = E // 16
NCH1 = EPT1 // CH
NSL = NPAD // 16

_MESH = dict(
    mesh=plsc.VectorSubcoreMesh(core_axis_name="c", subcore_axis_name="s"),
    compiler_params=pltpu.CompilerParams(needs_layout_passes=False),
)
_INT_MIN = -2147483648


@functools.partial(
    pl.kernel,
    out_type=[
        jax.ShapeDtypeStruct((E,), jnp.float32),
        jax.ShapeDtypeStruct((E,), jnp.int32),
        jax.ShapeDtypeStruct((NTILES * NPAD,), jnp.float32),
    ],
    scratch_types=[
        pltpu.VMEM((NPAD,), jnp.float32),
        pltpu.VMEM((EPT,), jnp.int32),
        pltpu.VMEM((EPT,), jnp.int32),
        pltpu.VMEM((EPT,), jnp.float32),
        pltpu.VMEM((EPT,), jnp.float32),
        pltpu.VMEM((EPT,), jnp.int32),
        pltpu.VMEM((NPAD,), jnp.float32),
    ],
    **_MESH,
)
def _sc_edge_prep(nv_hbm, src_hbm, dst_hbm, ap_hbm,
                  alive_hbm, dred_hbm, deg_hbm,
                  nv_v, src_v, dst_v, ap_v, al_v, dr_v, deg_v):
    cid = lax.axis_index("c")
    sid = lax.axis_index("s")
    wid = cid * 16 + sid
    base = wid * EPT

    pltpu.sync_copy(nv_hbm, nv_v)
    pltpu.sync_copy(src_hbm.at[pl.ds(base, EPT)], src_v)
    pltpu.sync_copy(dst_hbm.at[pl.ds(base, EPT)], dst_v)
    pltpu.sync_copy(ap_hbm.at[pl.ds(base, EPT)], ap_v)

    def zero(i, _):
        deg_v[pl.ds(i * 16, 16)] = jnp.zeros((16,), jnp.float32)
        return _
    lax.fori_loop(0, NPAD // 16, zero, 0)

    def grp(c, _):
        sl = pl.ds(c * 16, 16)
        sv = src_v[sl]
        dv = dst_v[sl]
        al = ap_v[sl] * plsc.load_gather(nv_v, [sv]) * plsc.load_gather(nv_v, [dv])
        al_v[sl] = al
        dr_v[sl] = jnp.where(al > 0.0, dv, DUMMY)
        plsc.addupdate_scatter(deg_v, [dv], al)
        return _
    lax.fori_loop(0, EPT // 16, grp, 0)

    pltpu.sync_copy(al_v, alive_hbm.at[pl.ds(base, EPT)])
    pltpu.sync_copy(dr_v, dred_hbm.at[pl.ds(base, EPT)])
    pltpu.sync_copy(deg_v, deg_hbm.at[pl.ds(wid * NPAD, NPAD)])


_NBUF = 5
_NBFA = 2


@functools.partial(
    pl.kernel,
    out_type=[jax.ShapeDtypeStruct((2, NPAD, D), jnp.float32)],
    scratch_types=[
        pltpu.VMEM((EPT,), jnp.int32),
        pltpu.VMEM((EPT,), jnp.int32),
        *[pltpu.VMEM((CH, D), jnp.float32) for _ in range(_NBFA)],
        *[pltpu.VMEM((CH,), jnp.int32) for _ in range(_NBFA)],
        *[pltpu.SemaphoreType.DMA for _ in range(2 * _NBFA)],
        pltpu.VMEM_SHARED((NPAD, D), jnp.float32),
    ],
    **_MESH,
)
def _sc_feat_agg(y_hbm, src_hbm, dred_hbm, zr_hbm, agg_hbm,
                 idxs, idxd, *rest):
    rows = rest[:_NBFA]
    idxb = rest[_NBFA:2 * _NBFA]
    semg = rest[2 * _NBFA:3 * _NBFA]
    sems = rest[3 * _NBFA:4 * _NBFA]
    acc_sp = rest[4 * _NBFA]
    cid = lax.axis_index("c")
    sid = lax.axis_index("s")
    wid = cid * 16 + sid

    pltpu.sync_copy(src_hbm.at[pl.ds(wid * EPT, EPT)], idxs)
    pltpu.sync_copy(dred_hbm.at[pl.ds(wid * EPT, EPT)], idxd)
    for t in range(NSL // CH):
        pltpu.sync_copy(zr_hbm, acc_sp.at[pl.ds(sid * NSL + t * CH, CH)])
    plsc.subcore_barrier()

    def do_chunks(g, nb):
        gath = [pltpu.async_copy(
            y_hbm.at[idxs.at[pl.ds((g * _NBFA + b) * CH, CH)]],
            rows[b], semg[b]) for b in range(nb)]
        scat = []
        for b in range(nb):
            c = g * _NBFA + b
            for j in range(CH // 16):
                idxb[b][pl.ds(j * 16, 16)] = idxd[pl.ds(c * CH + j * 16, 16)]
            gath[b].wait()
            scat.append(pltpu.async_copy(rows[b], acc_sp.at[idxb[b]],
                                         sems[b], add=True))
        for b in range(nb):
            scat[b].wait()

    def group(g, _):
        do_chunks(g, _NBFA)
        return _

    lax.fori_loop(0, NCH // _NBFA, group, 0)
    do_chunks(NCH // _NBFA, NCH % _NBFA)
    plsc.subcore_barrier()
    pltpu.sync_copy(acc_sp.at[pl.ds(sid * NSL, NSL)],
                    agg_hbm.at[cid, pl.ds(sid * NSL, NSL)])


@functools.partial(
    pl.kernel,
    out_type=[jax.ShapeDtypeStruct((NPAD,), jnp.float32)],
    scratch_types=[
        pltpu.VMEM((NPAD,), jnp.float32),
        pltpu.VMEM((NPAD,), jnp.float32),
        pltpu.VMEM((EPT1,), jnp.float32),
        pltpu.VMEM((EPT1,), jnp.int32),
        pltpu.VMEM((EPT1,), jnp.int32),
        pltpu.VMEM((CH,), jnp.float32),
        *[pltpu.VMEM((CH,), jnp.float32) for _ in range(_NBUF)],
        *[pltpu.VMEM((CH,), jnp.int32) for _ in range(_NBUF)],
        *[pltpu.SemaphoreType.DMA for _ in range(_NBUF)],
        pltpu.VMEM((NSL,), jnp.float32),
        pltpu.VMEM((NSL,), jnp.float32),
        pltpu.VMEM((NSL,), jnp.float32),
        pltpu.VMEM((NSL,), jnp.float32),
        pltpu.VMEM((NSL,), jnp.float32),
        pltpu.VMEM_SHARED((NPAD,), jnp.float32),
        pltpu.VMEM_SHARED((NPAD,), jnp.float32),
    ],
    **_MESH,
)
def _sc_sprop(s0_hbm, dinv_hbm, dinv2_hbm, src_hbm, dst_hbm, alive_hbm,
              s_hbm,
              dinv_v, s_v, coef_v, src_v, dst_v, alb, *rest):
    valb = rest[:_NBUF]
    idxb = rest[_NBUF:2 * _NBUF]
    sems = rest[2 * _NBUF:3 * _NBUF]
    s0b, d2b, propb, snewb, zb, prop_sp, s_sp = rest[3 * _NBUF:]
    cid = lax.axis_index("c")
    sid = lax.axis_index("s")

    @pl.when(cid == 0)
    def _body():
        nsl = pl.ds(sid * NSL, NSL)
        base = sid * EPT1
        pltpu.sync_copy(dinv_hbm, dinv_v)
        pltpu.sync_copy(s0_hbm, s_v)
        pltpu.sync_copy(s0_hbm.at[nsl], s0b)
        pltpu.sync_copy(dinv2_hbm.at[nsl], d2b)
        pltpu.sync_copy(src_hbm.at[pl.ds(base, EPT1)], src_v)
        pltpu.sync_copy(dst_hbm.at[pl.ds(base, EPT1)], dst_v)

        def coef_chunk(c, _):
            pltpu.sync_copy(alive_hbm.at[pl.ds(base + c * CH, CH)], alb)
            for j in range(CH // 16):
                sl = pl.ds(c * CH + j * 16, 16)
                gs = plsc.load_gather(dinv_v, [src_v[sl]])
                gd = plsc.load_gather(dinv_v, [dst_v[sl]])
                coef_v[sl] = alb[pl.ds(j * 16, 16)] * gs * gd
            return _

        lax.fori_loop(0, NCH1, coef_chunk, 0)
        for i in range(NSL // 16):
            zb[pl.ds(i * 16, 16)] = jnp.zeros((16,), jnp.float32)
        pltpu.sync_copy(zb, prop_sp.at[nsl])
        plsc.subcore_barrier()

        for h in range(HOPS):
            def group(g, _):
                scat = []
                for b in range(_NBUF):
                    c = g * _NBUF + b
                    for j in range(CH // 16):
                        sl = pl.ds(c * CH + j * 16, 16)
                        sv = plsc.load_gather(s_v, [src_v[sl]])
                        valb[b][pl.ds(j * 16, 16)] = sv * coef_v[sl]
                        idxb[b][pl.ds(j * 16, 16)] = dst_v[sl]
                    scat.append(pltpu.async_copy(
                        valb[b], prop_sp.at[idxb[b]], sems[b], add=True))
                for b in range(_NBUF):
                    scat[b].wait()
                return _

            lax.fori_loop(0, NCH1 // _NBUF, group, 0)
            plsc.subcore_barrier()
            pltpu.sync_copy(prop_sp.at[nsl], propb)
            for i in range(NSL // 16):
                sl16 = pl.ds(i * 16, 16)
                sv = s_v[pl.ds(sid * NSL + i * 16, 16)]
                snewb[sl16] = ALPHA * s0b[sl16] + (1.0 - ALPHA) * (
                    propb[sl16] + sv * d2b[sl16])
            if h < HOPS - 1:
                pltpu.sync_copy(snewb, s_sp.at[nsl])
                pltpu.sync_copy(zb, prop_sp.at[nsl])
                plsc.subcore_barrier()
                pltpu.sync_copy(s_sp, s_v)
            else:
                pltpu.sync_copy(snewb, s_hbm.at[nsl])


_RB = 1024
_NRB = NPAD // _RB


def _tc_matmul(x, W):
    def body(x_ref, w_ref, o_ref):
        o_ref[...] = jnp.dot(x_ref[...], w_ref[...],
                             preferred_element_type=jnp.float32)
    return pl.pallas_call(
        body,
        grid=(_NRB,),
        in_specs=[pl.BlockSpec((_RB, D), lambda i: (i, 0)),
                  pl.BlockSpec((D, D), lambda i: (0, 0))],
        out_specs=pl.BlockSpec((_RB, D), lambda i: (i, 0)),
        out_shape=jax.ShapeDtypeStruct((NPAD, D), jnp.float32),
    )(x, W)


def _tc_mid(deg_p, xw):
    def body(dp_ref, xw_ref, y_ref, di_ref, d2_ref):
        deg = jnp.sum(dp_ref[...], axis=0) + 1.0
        dinv = lax.rsqrt(deg)
        di_ref[...] = dinv
        d2_ref[...] = 1.0 / deg
        y_ref[...] = xw_ref[...] * dinv
    return pl.pallas_call(
        body,
        grid=(_NRB,),
        in_specs=[pl.BlockSpec((NTILES, _RB, 1), lambda i: (0, i, 0)),
                  pl.BlockSpec((_RB, D), lambda i: (i, 0))],
        out_specs=[pl.BlockSpec((_RB, D), lambda i: (i, 0)),
                   pl.BlockSpec((_RB, 1), lambda i: (i, 0)),
                   pl.BlockSpec((_RB, 1), lambda i: (i, 0))],
        out_shape=[jax.ShapeDtypeStruct((NPAD, D), jnp.float32),
                   jax.ShapeDtypeStruct((NPAD, 1), jnp.float32),
                   jax.ShapeDtypeStruct((NPAD, 1), jnp.float32)],
    )(deg_p, xw)


def _tc_post(agg_p, xw, dinv, dinv2, nv, b, p):
    def body(ag_ref, xw_ref, di_ref, d2_ref, nv_ref, b_ref, p_ref,
             xn_ref, s0_ref):
        agg = ag_ref[0] + ag_ref[1]
        xn = jax.nn.relu(agg * di_ref[...] + xw_ref[...] * d2_ref[...]
                         + b_ref[...]) * nv_ref[...]
        xn_ref[...] = xn
        s0_ref[...] = jnp.dot(xn, p_ref[...],
                              preferred_element_type=jnp.float32)
    return pl.pallas_call(
        body,
        grid=(_NRB,),
        in_specs=[pl.BlockSpec((2, _RB, D), lambda i: (0, i, 0)),
                  pl.BlockSpec((_RB, D), lambda i: (i, 0)),
                  pl.BlockSpec((_RB, 1), lambda i: (i, 0)),
                  pl.BlockSpec((_RB, 1), lambda i: (i, 0)),
                  pl.BlockSpec((_RB, 1), lambda i: (i, 0)),
                  pl.BlockSpec((1, D), lambda i: (0, 0)),
                  pl.BlockSpec((D, 1), lambda i: (0, 0))],
        out_specs=[pl.BlockSpec((_RB, D), lambda i: (i, 0)),
                   pl.BlockSpec((_RB, 1), lambda i: (i, 0))],
        out_shape=[jax.ShapeDtypeStruct((NPAD, D), jnp.float32),
                   jax.ShapeDtypeStruct((NPAD, 1), jnp.float32)],
    )(agg_p, xw, dinv, dinv2, nv, b, p)


def _tc_pool(s, nv, xn, k):
    def body(s_ref, nv_ref, xn_ref, nvn_ref, xg_ref):
        s_val = s_ref[...]
        bits = lax.bitcast_convert_type(s_val, jnp.int32)
        key = bits ^ ((bits >> 31) & jnp.int32(0x7FFFFFFF))
        key = jnp.where(nv_ref[...] > 0.0, key, jnp.int32(_INT_MIN))

        def step(i, cur_u):
            cand_u = cur_u | (jnp.int32(1) << (31 - i))
            cand_key = cand_u ^ _INT_MIN
            cnt = jnp.sum((key >= cand_key).astype(jnp.int32))
            return jnp.where(cnt >= k, cand_u, cur_u)

        cur_u = lax.fori_loop(0, 32, step, jnp.int32(0))
        t_key = cur_u ^ _INT_MIN
        nvn = (key >= t_key).astype(jnp.float32)
        nvn_ref[...] = nvn
        xg_ref[...] = xn_ref[...] * jnp.tanh(s_val) * nvn
    return pl.pallas_call(
        body,
        out_shape=[jax.ShapeDtypeStruct((NPAD, 1), jnp.float32),
                   jax.ShapeDtypeStruct((NPAD, D), jnp.float32)],
    )(s, nv, xn)


def _tc_readout(xg, nv, batch):
    def body(xg_ref, nv_ref, b_ref, mx_ref, sm_ref, cnt_ref):
        pid = pl.program_id(0)

        @pl.when(pid == 0)
        def _():
            mx_ref[...] = jnp.full((NG, D), -jnp.inf, jnp.float32)
            sm_ref[...] = jnp.zeros((NG, D), jnp.float32)
            cnt_ref[...] = jnp.zeros((NG, 1), jnp.float32)

        bid = b_ref[...]
        nvb = nv_ref[...]
        xgb = xg_ref[...]
        gids = lax.broadcasted_iota(jnp.int32, (1, NG), 1)
        oh = (bid == gids).astype(jnp.float32) * nvb
        dn = (((0,), (0,)), ((), ()))
        sm_ref[...] += lax.dot_general(oh, xgb, dn,
                                       preferred_element_type=jnp.float32)
        cnt_ref[...] += lax.dot_general(oh, nvb, dn,
                                        preferred_element_type=jnp.float32)

        def gstep(g, _):
            m = jnp.max(jnp.where((bid == g) & (nvb > 0.0), xgb, -jnp.inf),
                        axis=0, keepdims=True)
            mx_ref[pl.ds(g, 1), :] = jnp.maximum(mx_ref[pl.ds(g, 1), :], m)
            return _

        lax.fori_loop(0, NG, gstep, 0)

    return pl.pallas_call(
        body,
        grid=(_NRB,),
        in_specs=[pl.BlockSpec((_RB, D), lambda i: (i, 0)),
                  pl.BlockSpec((_RB, 1), lambda i: (i, 0)),
                  pl.BlockSpec((_RB, 1), lambda i: (i, 0))],
        out_specs=[pl.BlockSpec((NG, D), lambda i: (0, 0)),
                   pl.BlockSpec((NG, D), lambda i: (0, 0)),
                   pl.BlockSpec((NG, 1), lambda i: (0, 0))],
        out_shape=[jax.ShapeDtypeStruct((NG, D), jnp.float32),
                   jax.ShapeDtypeStruct((NG, D), jnp.float32),
                   jax.ShapeDtypeStruct((NG, 1), jnp.float32)],
    )(xg, nv, batch)


def _tc_head(mxs, sms, cnts, Wl1, bl1, Wl2, bl2, Wl3, bl3):
    def body(m1, m2, m3, m4, s1, s2, s3, s4, c1, c2, c3, c4,
             w1, b1, w2, b2, w3, b3, o_ref):
        h = jnp.zeros((NG, 2 * D), jnp.float32)
        for m_ref, s_ref, c_ref in ((m1, s1, c1), (m2, s2, c2),
                                    (m3, s3, c3), (m4, s4, c4)):
            mx = m_ref[...]
            mx = jnp.where(jnp.isfinite(mx), mx, 0.0)
            mean = s_ref[...] / jnp.maximum(c_ref[...], 1.0)
            h = h + jnp.concatenate([mx, mean], axis=1)
        z = jax.nn.relu(jnp.dot(h, w1[...],
                                preferred_element_type=jnp.float32) + b1[...])
        z = jax.nn.relu(jnp.dot(z, w2[...],
                                preferred_element_type=jnp.float32) + b2[...])
        z = jnp.dot(z, w3[...], preferred_element_type=jnp.float32) + b3[...]
        zm = z - jnp.max(z, axis=1, keepdims=True)
        o_ref[...] = zm - jnp.log(jnp.sum(jnp.exp(zm), axis=1, keepdims=True))

    args = list(mxs) + list(sms) + list(cnts) + [
        Wl1, bl1.reshape(1, -1), Wl2, bl2.reshape(1, -1), Wl3, bl3.reshape(1, -1)]
    return pl.pallas_call(
        body,
        out_shape=jax.ShapeDtypeStruct((NG, 10), jnp.float32),
    )(*args)


def kernel(x, edge_index, batch, W1, b1, p1, W2, b2, p2, W3, b3, p3,
           W4, b4, p4, Wl1, bl1, Wl2, bl2, Wl3, bl3):
    src = edge_index[0]
    dst = edge_index[1]
    xs = jnp.pad(x, ((0, NPAD - N), (0, 0)))
    batchp = jnp.pad(batch, (0, NPAD - N)).reshape(NPAD, 1)
    nv_col = jnp.pad(jnp.ones((N, 1), jnp.float32), ((0, NPAD - N), (0, 0)))
    alive = jnp.ones((E,), jnp.float32)
    zr = jnp.zeros((CH, D), jnp.float32)

    Ws = (W1, W2, W3, W4)
    bs = (b1, b2, b3, b4)
    ps = (p1, p2, p3, p4)
    mxs, sms, cnts = [], [], []

    for l in range(4):
        nv_flat = nv_col.reshape(NPAD)
        alive, dred, deg_p = _sc_edge_prep(nv_flat, src, dst, alive)
        xw = _tc_matmul(xs, Ws[l])
        y, dinv, dinv2 = _tc_mid(deg_p.reshape(NTILES, NPAD, 1), xw)
        (agg_p,) = _sc_feat_agg(y, src, dred, zr)
        xn, s0 = _tc_post(agg_p, xw, dinv, dinv2, nv_col,
                          bs[l].reshape(1, D), ps[l])
        (s,) = _sc_sprop(s0.reshape(NPAD), dinv.reshape(NPAD),
                         dinv2.reshape(NPAD), src, dst, alive)
        nv_col, xg = _tc_pool(s.reshape(NPAD, 1), nv_col, xn, KS[l])
        mx, sm, cnt = _tc_readout(xg, nv_col, batchp)
        mxs.append(mx); sms.append(sm); cnts.append(cnt)
        xs = xg

    return _tc_head(mxs, sms, cnts, Wl1, bl1, Wl2, bl2, Wl3, bl3)
